# baseline (speedup 1.0000x reference)
import functools

import jax
import jax.numpy as jnp
from jax import lax
from jax.experimental import pallas as pl
from jax.experimental.pallas import tpu as pltpu
from jax.experimental.pallas import tpu_sc as plsc

B = 16384
D = 64
NW = 32
BW = B // NW
NCHUNK = 4
CW = BW // NCHUNK
L = 16


def _sc_partials(h, t, r, ent, rel):
    mesh = plsc.VectorSubcoreMesh(core_axis_name="c", subcore_axis_name="s")

    @functools.partial(
        pl.kernel,
        mesh=mesh,
        out_type=jax.ShapeDtypeStruct((3, NW, BW, L), jnp.float32),
        compiler_params=pltpu.CompilerParams(use_tc_tiling_on_sc=False),
        scratch_types=[
            pltpu.VMEM((NCHUNK, CW), jnp.int32),
            pltpu.VMEM((NCHUNK, CW), jnp.int32),
            pltpu.VMEM((NCHUNK, CW), jnp.int32),
            pltpu.VMEM((BW, D), jnp.float32),
            pltpu.VMEM((BW, D), jnp.float32),
            pltpu.VMEM((BW, D), jnp.float32),
            pltpu.VMEM((BW, L), jnp.float32),
            pltpu.VMEM((BW, L), jnp.float32),
            pltpu.VMEM((BW, L), jnp.float32),
            pltpu.SemaphoreType.DMA,
            pltpu.SemaphoreType.DMA,
            pltpu.SemaphoreType.DMA,
        ],
    )
    def sc_kernel(h_hbm, t_hbm, r_hbm, ent_hbm, rel_hbm, out_hbm,
                  idx_h, idx_t, idx_r, rows_h, rows_t, rows_r,
                  part_h, part_t, part_r, sem_h, sem_t, sem_r):
        wid = lax.axis_index("s") * 2 + lax.axis_index("c")
        base = wid * BW

        for c in range(NCHUNK):
            off = base + c * CW
            pltpu.sync_copy(h_hbm.at[pl.ds(off, CW)], idx_h.at[c])
            pltpu.sync_copy(t_hbm.at[pl.ds(off, CW)], idx_t.at[c])
            pltpu.sync_copy(r_hbm.at[pl.ds(off, CW)], idx_r.at[c])

        copies = []
        for table, idx, rows, sem in (
            (ent_hbm, idx_h, rows_h, sem_h),
            (ent_hbm, idx_t, rows_t, sem_t),
            (rel_hbm, idx_r, rows_r, sem_r),
        ):
            for c in range(NCHUNK):
                copies.append(pltpu.async_copy(
                    table.at[idx.at[c]], rows.at[pl.ds(c * CW, CW)], sem))

        def fold_rows(rows, part):
            def body(i, carry):
                v0 = rows[i, pl.ds(0, L)]
                v1 = rows[i, pl.ds(L, L)]
                v2 = rows[i, pl.ds(2 * L, L)]
                v3 = rows[i, pl.ds(3 * L, L)]
                part[i, :] = v0 * v0 + v1 * v1 + v2 * v2 + v3 * v3
                return carry
            lax.fori_loop(0, BW, body, 0, unroll=8)

        for k, (rows, part) in enumerate(
                ((rows_h, part_h), (rows_t, part_t), (rows_r, part_r))):
            for c in range(NCHUNK):
                copies[k * NCHUNK + c].wait()
            fold_rows(rows, part)
            pltpu.sync_copy(part, out_hbm.at[k, wid])

    return sc_kernel(h, t, r, ent, rel)


def _tc_combine(p):
    def tc_kernel(p_ref, o_ref):
        col = lax.broadcasted_iota(jnp.int32, (128, 8), 0) // 16
        grp = lax.broadcasted_iota(jnp.int32, (128, 8), 1)
        m = (col == grp).astype(jnp.float32)
        sh = jnp.dot(p_ref[0], m, preferred_element_type=jnp.float32)
        st = jnp.dot(p_ref[1], m, preferred_element_type=jnp.float32)
        sr = jnp.dot(p_ref[2], m, preferred_element_type=jnp.float32)
        d = jnp.sqrt(sh) - jnp.sqrt(st) + jnp.sqrt(sr)
        o_ref[...] = jnp.sqrt(jnp.sum(d * d)).reshape(1, 1)

    return pl.pallas_call(
        tc_kernel,
        out_shape=jax.ShapeDtypeStruct((1, 1), jnp.float32),
    )(p)


def kernel(h, r, t, emb_entity, emb_relation, norm_p):
    parts = _sc_partials(h, t, r, emb_entity, emb_relation)
    parts = parts.reshape(3, B * L // 128, 128)
    out = _tc_combine(parts)[0, 0]
    pf = jnp.asarray(norm_p, jnp.float32)
    return out * (pf / pf)
